# baseline (speedup 1.0000x reference)
import functools

import jax
import jax.numpy as jnp
from jax import lax
from jax.experimental import pallas as pl
from jax.experimental.pallas import tpu as pltpu
from jax.experimental.pallas import tpu_sc as plsc

NC = 2
NS = 16
L = 16
GPAD = 112
NJ = 8
BLK = 16


def _sload(ref, i):
    return ref[pl.ds(i, L)][0]


def _searchsorted_right(ends_v, row, b):
    def step(k, lo):
        sz = b >> (k + 1)
        probe = lo + sz - 1
        go = _sload(ends_v, probe) <= row
        return lo + jnp.where(go, sz, 0)

    return lax.fori_loop(0, b.bit_length() - 1, step, jnp.int32(0))


def _hop(g, e, row, ends_v, nxt_v):
    adv = row >= e
    gn = _sload(nxt_v, g)
    en = _sload(ends_v, gn)
    return jnp.where(adv, gn, g), jnp.where(adv, en, e)


def _make_pass1(n, d, b, rpw, chunk, ring):
    d2 = 2 * d
    nfull = (rpw // chunk) // ring * ring
    rem = rpw - nfull * chunk
    nblk = chunk // BLK
    assert chunk % BLK == 0 and rem % 8 == 0
    mesh = plsc.VectorSubcoreMesh(core_axis_name="c", subcore_axis_name="s")

    @functools.partial(
        pl.kernel,
        out_type=jax.ShapeDtypeStruct((NC * NS, GPAD, d2), jnp.float32),
        mesh=mesh,
        scratch_types=(
            [pltpu.VMEM((chunk, d), jnp.float32)] * ring
            + [
                pltpu.VMEM((GPAD, d2), jnp.float32),
                pltpu.VMEM((b + L,), jnp.int32),
                pltpu.VMEM((b + L,), jnp.int32),
                pltpu.VMEM((2 * L,), jnp.int32),
            ]
            + [pltpu.SemaphoreType.DMA] * ring
        ),
    )
    def pass1(tensor_hbm, ends_hbm, nxt_hbm, part_hbm, *scr):
        bufs = scr[:ring]
        acc, ends_v, nxt_v, ptr = scr[ring:ring + 4]
        sems = scr[ring + 4:]
        cid = lax.axis_index("c")
        sid = lax.axis_index("s")
        wid = cid * NS + sid
        zero16 = jnp.zeros((L,), jnp.float32)

        pltpu.sync_copy(ends_hbm, ends_v.at[pl.ds(0, b)])
        pltpu.sync_copy(nxt_hbm, nxt_v.at[pl.ds(0, b)])
        ptr[pl.ds(0, L)] = jnp.zeros((L,), jnp.int32)
        ptr[pl.ds(L, L)] = jnp.zeros((L,), jnp.int32)

        def arow(r, carry):
            for j in range(d2 // L):
                acc[r, pl.ds(L * j, L)] = zero16
            return carry

        lax.fori_loop(0, GPAD, arow, 0)

        row0 = wid * rpw
        gstart = _searchsorted_right(ends_v, row0, b)
        g0 = pl.multiple_of(jnp.minimum(gstart & ~7, b - GPAD), 8)

        def row_update(buf, rloc, row, gR, eR):
            gR, eR = _hop(gR, eR, row, ends_v, nxt_v)
            gl = jnp.minimum(gR - g0, GPAD - 1)
            for j in range(NJ):
                v = buf[rloc, pl.ds(L * j, L)]
                plsc.addupdate(acc.at[gl, pl.ds(L * j, L)], v)
                plsc.addupdate(acc.at[gl, pl.ds(d + L * j, L)], v * v)
            return gR, eR

        def blocks_body(buf, base, carry):
            for k in range(nblk):
                gP, eP = carry
                rloc = k * BLK
                rb = base + rloc
                boundary = rb + BLK > eP
                glP = jnp.minimum(gP - g0, GPAD - 1)

                @pl.when(jnp.logical_not(boundary))
                def _fast(rloc=rloc, glP=glP):
                    for j in range(NJ):
                        bs = buf[rloc, pl.ds(L * j, L)]
                        bq = bs * bs
                        for r in range(1, BLK):
                            v = buf[rloc + r, pl.ds(L * j, L)]
                            bs = bs + v
                            bq = bq + v * v
                        plsc.addupdate(acc.at[glP, pl.ds(L * j, L)], bs)
                        plsc.addupdate(acc.at[glP, pl.ds(d + L * j, L)], bq)

                @pl.when(boundary)
                def _slow(rloc=rloc, rb=rb, gP=gP, eP=eP):
                    def srow(r, c3):
                        return row_update(buf, rloc + r, rb + r, c3[0], c3[1])

                    gR, eR = lax.fori_loop(0, BLK, srow, (gP, eP))
                    ptr[pl.ds(0, L)] = jnp.full((L,), gR, jnp.int32)
                    ptr[pl.ds(L, L)] = jnp.full((L,), eR, jnp.int32)

                gS = _sload(ptr, 0)
                eS = _sload(ptr, L)
                gP = jnp.where(boundary, gS, gP)
                eP = jnp.where(boundary, eS, eP)
                carry = _hop(gP, eP, rb + BLK, ends_v, nxt_v)
            return carry

        e0 = _sload(ends_v, gstart)
        carry = (gstart, e0)

        for i in range(ring):
            pltpu.async_copy(tensor_hbm.at[pl.ds(row0 + i * chunk, chunk)],
                             bufs[i], sems[i])

        def do_group(p, carry):
            base = row0 + ring * p * chunk
            for h in range(ring):
                pltpu.make_async_copy(tensor_hbm.at[pl.ds(0, chunk)],
                                      bufs[h], sems[h]).wait()

                @pl.when(ring * p + ring + h < nfull)
                def _prefetch(h=h, base=base):
                    pltpu.async_copy(
                        tensor_hbm.at[pl.ds(base + (ring + h) * chunk, chunk)],
                        bufs[h], sems[h])

                carry = blocks_body(bufs[h], base + h * chunk, carry)
            return carry

        carry = lax.fori_loop(0, nfull // ring, do_group, carry)
        off = 0
        while off < rem:
            sz = min(chunk, rem - off)
            base = row0 + nfull * chunk + off
            pltpu.sync_copy(tensor_hbm.at[pl.ds(base, sz)],
                            bufs[0].at[pl.ds(0, sz)])

            def tail_row(r, c2, base=base):
                return row_update(bufs[0], r, base + r, c2[0], c2[1])

            carry = lax.fori_loop(0, sz, tail_row, carry)
            off += sz

        pltpu.sync_copy(acc, part_hbm.at[wid])

    return pass1


def _make_pass2(n, d, b, rpw, chunk, ring):
    nfull = (rpw // chunk) // ring * ring
    rem = rpw - nfull * chunk
    nblk = chunk // BLK
    assert chunk % BLK == 0 and rem % 8 == 0
    mesh = plsc.VectorSubcoreMesh(core_axis_name="c", subcore_axis_name="s")

    @functools.partial(
        pl.kernel,
        out_type=jax.ShapeDtypeStruct((n, d), jnp.float32),
        mesh=mesh,
        scratch_types=(
            [pltpu.VMEM((chunk, d), jnp.float32)] * (2 * ring)
            + [
                pltpu.VMEM((GPAD, d), jnp.float32),
                pltpu.VMEM((GPAD, d), jnp.float32),
                pltpu.VMEM((b + L,), jnp.int32),
                pltpu.VMEM((b + L,), jnp.int32),
                pltpu.VMEM((2 * L,), jnp.int32),
            ]
            + [pltpu.SemaphoreType.DMA] * (2 * ring)
        ),
    )
    def pass2(tensor_hbm, ends_hbm, nxt_hbm, alpha_hbm, beta_hbm, out_hbm,
              *scr):
        bufs = scr[:ring]
        obufs = scr[ring:2 * ring]
        al, be, ends_v, nxt_v, ptr = scr[2 * ring:2 * ring + 5]
        sems = scr[2 * ring + 5:3 * ring + 5]
        osems = scr[3 * ring + 5:]
        cid = lax.axis_index("c")
        sid = lax.axis_index("s")
        wid = cid * NS + sid

        pltpu.sync_copy(ends_hbm, ends_v.at[pl.ds(0, b)])
        pltpu.sync_copy(nxt_hbm, nxt_v.at[pl.ds(0, b)])
        ptr[pl.ds(0, L)] = jnp.zeros((L,), jnp.int32)
        ptr[pl.ds(L, L)] = jnp.zeros((L,), jnp.int32)
        row0 = wid * rpw
        gstart = _searchsorted_right(ends_v, row0, b)
        g0c = pl.multiple_of(jnp.minimum(gstart & ~7, b - GPAD), 8)
        pltpu.sync_copy(alpha_hbm.at[pl.ds(g0c, GPAD)], al)
        pltpu.sync_copy(beta_hbm.at[pl.ds(g0c, GPAD)], be)

        def row_apply(buf, obuf, rloc, row, gR, eR):
            gR, eR = _hop(gR, eR, row, ends_v, nxt_v)
            gl = jnp.minimum(gR - g0c, GPAD - 1)
            for j in range(NJ):
                v = buf[rloc, pl.ds(L * j, L)]
                a = al[gl, pl.ds(L * j, L)]
                bb = be[gl, pl.ds(L * j, L)]
                obuf[rloc, pl.ds(L * j, L)] = v * a + bb
            return gR, eR

        def blocks_body(buf, obuf, base, carry):
            for k in range(nblk):
                gP, eP = carry
                rloc = k * BLK
                rb = base + rloc
                boundary = rb + BLK > eP
                glP = jnp.minimum(gP - g0c, GPAD - 1)

                @pl.when(jnp.logical_not(boundary))
                def _fast(rloc=rloc, glP=glP):
                    a = [al[glP, pl.ds(L * j, L)] for j in range(NJ)]
                    bb = [be[glP, pl.ds(L * j, L)] for j in range(NJ)]
                    for r in range(BLK):
                        for j in range(NJ):
                            v = buf[rloc + r, pl.ds(L * j, L)]
                            obuf[rloc + r, pl.ds(L * j, L)] = v * a[j] + bb[j]

                @pl.when(boundary)
                def _slow(rloc=rloc, rb=rb, gP=gP, eP=eP):
                    def srow(r, c3):
                        return row_apply(buf, obuf, rloc + r, rb + r,
                                         c3[0], c3[1])

                    gR, eR = lax.fori_loop(0, BLK, srow, (gP, eP))
                    ptr[pl.ds(0, L)] = jnp.full((L,), gR, jnp.int32)
                    ptr[pl.ds(L, L)] = jnp.full((L,), eR, jnp.int32)

                gS = _sload(ptr, 0)
                eS = _sload(ptr, L)
                gP = jnp.where(boundary, gS, gP)
                eP = jnp.where(boundary, eS, eP)
                carry = _hop(gP, eP, rb + BLK, ends_v, nxt_v)
            return carry

        e0 = _sload(ends_v, gstart)
        carry = (gstart, e0)

        for i in range(ring):
            pltpu.async_copy(tensor_hbm.at[pl.ds(row0 + i * chunk, chunk)],
                             bufs[i], sems[i])

        def do_group(p, carry):
            base = row0 + ring * p * chunk
            for h in range(ring):
                cbase = base + h * chunk
                pltpu.make_async_copy(tensor_hbm.at[pl.ds(0, chunk)],
                                      bufs[h], sems[h]).wait()

                @pl.when(ring * p + ring + h < nfull)
                def _prefetch(h=h, base=base):
                    pltpu.async_copy(
                        tensor_hbm.at[pl.ds(base + (ring + h) * chunk, chunk)],
                        bufs[h], sems[h])

                @pl.when(p > 0)
                def _drain_out(h=h):
                    pltpu.make_async_copy(obufs[h],
                                          out_hbm.at[pl.ds(0, chunk)],
                                          osems[h]).wait()

                carry = blocks_body(bufs[h], obufs[h], cbase, carry)
                pltpu.async_copy(obufs[h], out_hbm.at[pl.ds(cbase, chunk)],
                                 osems[h])
            return carry

        carry = lax.fori_loop(0, nfull // ring, do_group, carry)
        for h in range(ring):
            pltpu.make_async_copy(obufs[h], out_hbm.at[pl.ds(0, chunk)],
                                  osems[h]).wait()
        off = 0
        while off < rem:
            sz = min(chunk, rem - off)
            base = row0 + nfull * chunk + off
            pltpu.sync_copy(tensor_hbm.at[pl.ds(base, sz)],
                            bufs[0].at[pl.ds(0, sz)])

            def tail_row(r, c2, base=base):
                return row_apply(bufs[0], obufs[0], r, base + r, c2[0], c2[1])

            carry = lax.fori_loop(0, sz, tail_row, carry)
            pltpu.sync_copy(obufs[0].at[pl.ds(0, sz)],
                            out_hbm.at[pl.ds(base, sz)])
            off += sz

    return pass2


def _finalize_tc(partials, gtarg, counts, weight, bias, mean_scale, b):
    wg, d2 = partials.shape[0] * partials.shape[1], partials.shape[2]
    d = d2 // 2

    def body(part_ref, gt_ref, cnt_ref, w_ref, b_ref, ms_ref, alpha_ref, beta_ref):
        gidx = lax.broadcasted_iota(jnp.int32, (wg, b), 1)
        onehot = (gt_ref[...] == gidx).astype(jnp.float32)
        part2d = part_ref[...].reshape(wg, d2)
        s = lax.dot_general(onehot, part2d, (((0,), (0,)), ((), ())),
                            preferred_element_type=jnp.float32)
        sx = s[:, :d]
        sx2 = s[:, d:]
        invn = 1.0 / cnt_ref[...]
        mean = sx * invn
        c = mean * ms_ref[...]
        var = sx2 * invn - 2.0 * c * mean + c * c
        alpha = w_ref[...] * lax.rsqrt(var + 1e-6)
        alpha_ref[...] = alpha
        beta_ref[...] = b_ref[...] - alpha * c

    return pl.pallas_call(
        body,
        out_shape=[
            jax.ShapeDtypeStruct((b, d), jnp.float32),
            jax.ShapeDtypeStruct((b, d), jnp.float32),
        ],
    )(partials, gtarg, counts, weight, bias, mean_scale)


def kernel(graph, tensor, batch_list, weight, bias, mean_scale):
    n, d = tensor.shape
    b = batch_list.shape[0]
    w = NC * NS
    rpw = n // w
    assert n == w * rpw and rpw % 8 == 0

    ends = jnp.cumsum(batch_list.astype(jnp.int32))
    ne_idx = jnp.where(batch_list > 0, jnp.arange(b, dtype=jnp.int32), b - 1)
    sufmin = lax.cummin(ne_idx[::-1])[::-1]
    nxt = jnp.concatenate([sufmin[1:], jnp.full((1,), b - 1, jnp.int32)])
    partials = _make_pass1(n, d, b, rpw, 32, 4)(tensor, ends, nxt)
    g0 = jnp.searchsorted(ends, jnp.arange(w, dtype=jnp.int32) * rpw,
                          side='right').astype(jnp.int32)
    g0 = jnp.minimum(g0 & ~7, b - GPAD)
    gtarg = (g0[:, None] + jnp.arange(GPAD, dtype=jnp.int32)[None, :]).reshape(-1, 1)
    counts = batch_list.astype(jnp.float32).reshape(b, 1)
    alpha, beta = _finalize_tc(
        partials, gtarg, counts,
        weight.reshape(1, d), bias.reshape(1, d), mean_scale.reshape(1, d), b)
    return _make_pass2(n, d, b, rpw, 32, 4)(tensor, ends, nxt, alpha, beta)


# speedup vs baseline: 1.1080x; 1.1080x over previous
import functools

import jax
import jax.numpy as jnp
from jax import lax
from jax.experimental import pallas as pl
from jax.experimental.pallas import tpu as pltpu
from jax.experimental.pallas import tpu_sc as plsc

NC = 2
NS = 16
L = 16
GPAD = 112
NJ = 8
BLK = 16


def _sload(ref, i):
    return ref[pl.ds(i, L)][0]


def _searchsorted_right(ends_v, row, b):
    def step(k, lo):
        sz = b >> (k + 1)
        probe = lo + sz - 1
        go = _sload(ends_v, probe) <= row
        return lo + jnp.where(go, sz, 0)

    return lax.fori_loop(0, b.bit_length() - 1, step, jnp.int32(0))


def _hop(g, e, row, ends_v, nxt_v):
    adv = row >= e
    gn = _sload(nxt_v, g)
    en = _sload(ends_v, gn)
    return jnp.where(adv, gn, g), jnp.where(adv, en, e)


def _make_pass1(n, d, b, rpw, chunk, ring):
    d2 = 2 * d
    nfull = (rpw // chunk) // ring * ring
    rem = rpw - nfull * chunk
    nblk = chunk // BLK
    assert chunk % BLK == 0 and rem % 8 == 0
    mesh = plsc.VectorSubcoreMesh(core_axis_name="c", subcore_axis_name="s")

    @functools.partial(
        pl.kernel,
        out_type=jax.ShapeDtypeStruct((NC * NS, GPAD, d2), jnp.float32),
        mesh=mesh,
        scratch_types=(
            [pltpu.VMEM((chunk, d), jnp.float32)] * ring
            + [
                pltpu.VMEM((GPAD, d2), jnp.float32),
                pltpu.VMEM((b + L,), jnp.int32),
                pltpu.VMEM((b + L,), jnp.int32),
                pltpu.VMEM((2 * L,), jnp.int32),
            ]
            + [pltpu.SemaphoreType.DMA] * ring
        ),
    )
    def pass1(tensor_hbm, ends_hbm, nxt_hbm, part_hbm, *scr):
        bufs = scr[:ring]
        acc, ends_v, nxt_v, ptr = scr[ring:ring + 4]
        sems = scr[ring + 4:]
        cid = lax.axis_index("c")
        sid = lax.axis_index("s")
        wid = cid * NS + sid
        zero16 = jnp.zeros((L,), jnp.float32)

        pltpu.sync_copy(ends_hbm, ends_v.at[pl.ds(0, b)])
        pltpu.sync_copy(nxt_hbm, nxt_v.at[pl.ds(0, b)])
        ptr[pl.ds(0, L)] = jnp.zeros((L,), jnp.int32)
        ptr[pl.ds(L, L)] = jnp.zeros((L,), jnp.int32)

        def arow(r, carry):
            for j in range(d2 // L):
                acc[r, pl.ds(L * j, L)] = zero16
            return carry

        lax.fori_loop(0, GPAD, arow, 0)

        row0 = wid * rpw
        gstart = _searchsorted_right(ends_v, row0, b)
        g0 = pl.multiple_of(jnp.minimum(gstart & ~7, b - GPAD), 8)

        def row_update(buf, rloc, row, gR, eR):
            gR, eR = _hop(gR, eR, row, ends_v, nxt_v)
            gl = jnp.minimum(gR - g0, GPAD - 1)
            for j in range(NJ):
                v = buf[rloc, pl.ds(L * j, L)]
                plsc.addupdate(acc.at[gl, pl.ds(L * j, L)], v)
                plsc.addupdate(acc.at[gl, pl.ds(d + L * j, L)], v * v)
            return gR, eR

        def blocks_body(buf, base, carry):
            def do_block(k, c2):
                gP, eP = c2
                rloc = k * BLK
                rb = base + rloc
                boundary = rb + BLK > eP
                glP = jnp.minimum(gP - g0, GPAD - 1)

                @pl.when(jnp.logical_not(boundary))
                def _fast():
                    def frow(r, c4):
                        vs = [buf[rloc + r, pl.ds(L * j, L)]
                              for j in range(NJ)]
                        bs = tuple(c4[j] + vs[j] for j in range(NJ))
                        bq = tuple(c4[NJ + j] + vs[j] * vs[j]
                                   for j in range(NJ))
                        return bs + bq

                    res = lax.fori_loop(0, BLK, frow, (zero16,) * (2 * NJ))
                    for j in range(NJ):
                        plsc.addupdate(acc.at[glP, pl.ds(L * j, L)], res[j])
                        plsc.addupdate(acc.at[glP, pl.ds(d + L * j, L)],
                                       res[NJ + j])

                @pl.when(boundary)
                def _slow():
                    def srow(r, c3):
                        return row_update(buf, rloc + r, rb + r, c3[0], c3[1])

                    gR, eR = lax.fori_loop(0, BLK, srow, (gP, eP))
                    ptr[pl.ds(0, L)] = jnp.full((L,), gR, jnp.int32)
                    ptr[pl.ds(L, L)] = jnp.full((L,), eR, jnp.int32)

                gS = _sload(ptr, 0)
                eS = _sload(ptr, L)
                gP = jnp.where(boundary, gS, gP)
                eP = jnp.where(boundary, eS, eP)
                return _hop(gP, eP, rb + BLK, ends_v, nxt_v)

            return lax.fori_loop(0, nblk, do_block, carry)

        e0 = _sload(ends_v, gstart)
        carry = (gstart, e0)

        for i in range(ring):
            pltpu.async_copy(tensor_hbm.at[pl.ds(row0 + i * chunk, chunk)],
                             bufs[i], sems[i])

        def do_group(p, carry):
            base = row0 + ring * p * chunk
            for h in range(ring):
                pltpu.make_async_copy(tensor_hbm.at[pl.ds(0, chunk)],
                                      bufs[h], sems[h]).wait()

                @pl.when(ring * p + ring + h < nfull)
                def _prefetch(h=h, base=base):
                    pltpu.async_copy(
                        tensor_hbm.at[pl.ds(base + (ring + h) * chunk, chunk)],
                        bufs[h], sems[h])

                carry = blocks_body(bufs[h], base + h * chunk, carry)
            return carry

        carry = lax.fori_loop(0, nfull // ring, do_group, carry)
        off = 0
        while off < rem:
            sz = min(chunk, rem - off)
            base = row0 + nfull * chunk + off
            pltpu.sync_copy(tensor_hbm.at[pl.ds(base, sz)],
                            bufs[0].at[pl.ds(0, sz)])

            def tail_row(r, c2, base=base):
                return row_update(bufs[0], r, base + r, c2[0], c2[1])

            carry = lax.fori_loop(0, sz, tail_row, carry)
            off += sz

        pltpu.sync_copy(acc, part_hbm.at[wid])

    return pass1


def _make_pass2(n, d, b, rpw, chunk, ring):
    nfull = (rpw // chunk) // ring * ring
    rem = rpw - nfull * chunk
    nblk = chunk // BLK
    assert chunk % BLK == 0 and rem % 8 == 0
    mesh = plsc.VectorSubcoreMesh(core_axis_name="c", subcore_axis_name="s")

    @functools.partial(
        pl.kernel,
        out_type=jax.ShapeDtypeStruct((n, d), jnp.float32),
        mesh=mesh,
        scratch_types=(
            [pltpu.VMEM((chunk, d), jnp.float32)] * (2 * ring)
            + [
                pltpu.VMEM((GPAD, d), jnp.float32),
                pltpu.VMEM((GPAD, d), jnp.float32),
                pltpu.VMEM((b + L,), jnp.int32),
                pltpu.VMEM((b + L,), jnp.int32),
                pltpu.VMEM((2 * L,), jnp.int32),
            ]
            + [pltpu.SemaphoreType.DMA] * (2 * ring)
        ),
    )
    def pass2(tensor_hbm, ends_hbm, nxt_hbm, alpha_hbm, beta_hbm, out_hbm,
              *scr):
        bufs = scr[:ring]
        obufs = scr[ring:2 * ring]
        al, be, ends_v, nxt_v, ptr = scr[2 * ring:2 * ring + 5]
        sems = scr[2 * ring + 5:3 * ring + 5]
        osems = scr[3 * ring + 5:]
        cid = lax.axis_index("c")
        sid = lax.axis_index("s")
        wid = cid * NS + sid

        pltpu.sync_copy(ends_hbm, ends_v.at[pl.ds(0, b)])
        pltpu.sync_copy(nxt_hbm, nxt_v.at[pl.ds(0, b)])
        ptr[pl.ds(0, L)] = jnp.zeros((L,), jnp.int32)
        ptr[pl.ds(L, L)] = jnp.zeros((L,), jnp.int32)
        row0 = wid * rpw
        gstart = _searchsorted_right(ends_v, row0, b)
        g0c = pl.multiple_of(jnp.minimum(gstart & ~7, b - GPAD), 8)
        pltpu.sync_copy(alpha_hbm.at[pl.ds(g0c, GPAD)], al)
        pltpu.sync_copy(beta_hbm.at[pl.ds(g0c, GPAD)], be)

        def row_apply(buf, obuf, rloc, row, gR, eR):
            gR, eR = _hop(gR, eR, row, ends_v, nxt_v)
            gl = jnp.minimum(gR - g0c, GPAD - 1)
            for j in range(NJ):
                v = buf[rloc, pl.ds(L * j, L)]
                a = al[gl, pl.ds(L * j, L)]
                bb = be[gl, pl.ds(L * j, L)]
                obuf[rloc, pl.ds(L * j, L)] = v * a + bb
            return gR, eR

        def blocks_body(buf, obuf, base, carry):
            def do_block(k, c2):
                gP, eP = c2
                rloc = k * BLK
                rb = base + rloc
                boundary = rb + BLK > eP
                glP = jnp.minimum(gP - g0c, GPAD - 1)

                @pl.when(jnp.logical_not(boundary))
                def _fast():
                    a = [al[glP, pl.ds(L * j, L)] for j in range(NJ)]
                    bb = [be[glP, pl.ds(L * j, L)] for j in range(NJ)]

                    def frow(r, c4):
                        for j in range(NJ):
                            v = buf[rloc + r, pl.ds(L * j, L)]
                            obuf[rloc + r, pl.ds(L * j, L)] = v * a[j] + bb[j]
                        return c4

                    lax.fori_loop(0, BLK, frow, 0)

                @pl.when(boundary)
                def _slow():
                    def srow(r, c3):
                        return row_apply(buf, obuf, rloc + r, rb + r,
                                         c3[0], c3[1])

                    gR, eR = lax.fori_loop(0, BLK, srow, (gP, eP))
                    ptr[pl.ds(0, L)] = jnp.full((L,), gR, jnp.int32)
                    ptr[pl.ds(L, L)] = jnp.full((L,), eR, jnp.int32)

                gS = _sload(ptr, 0)
                eS = _sload(ptr, L)
                gP = jnp.where(boundary, gS, gP)
                eP = jnp.where(boundary, eS, eP)
                return _hop(gP, eP, rb + BLK, ends_v, nxt_v)

            return lax.fori_loop(0, nblk, do_block, carry)

        e0 = _sload(ends_v, gstart)
        carry = (gstart, e0)

        for i in range(ring):
            pltpu.async_copy(tensor_hbm.at[pl.ds(row0 + i * chunk, chunk)],
                             bufs[i], sems[i])

        def do_group(p, carry):
            base = row0 + ring * p * chunk
            for h in range(ring):
                cbase = base + h * chunk
                pltpu.make_async_copy(tensor_hbm.at[pl.ds(0, chunk)],
                                      bufs[h], sems[h]).wait()

                @pl.when(ring * p + ring + h < nfull)
                def _prefetch(h=h, base=base):
                    pltpu.async_copy(
                        tensor_hbm.at[pl.ds(base + (ring + h) * chunk, chunk)],
                        bufs[h], sems[h])

                @pl.when(p > 0)
                def _drain_out(h=h):
                    pltpu.make_async_copy(obufs[h],
                                          out_hbm.at[pl.ds(0, chunk)],
                                          osems[h]).wait()

                carry = blocks_body(bufs[h], obufs[h], cbase, carry)
                pltpu.async_copy(obufs[h], out_hbm.at[pl.ds(cbase, chunk)],
                                 osems[h])
            return carry

        carry = lax.fori_loop(0, nfull // ring, do_group, carry)
        for h in range(ring):
            pltpu.make_async_copy(obufs[h], out_hbm.at[pl.ds(0, chunk)],
                                  osems[h]).wait()
        off = 0
        while off < rem:
            sz = min(chunk, rem - off)
            base = row0 + nfull * chunk + off
            pltpu.sync_copy(tensor_hbm.at[pl.ds(base, sz)],
                            bufs[0].at[pl.ds(0, sz)])

            def tail_row(r, c2, base=base):
                return row_apply(bufs[0], obufs[0], r, base + r, c2[0], c2[1])

            carry = lax.fori_loop(0, sz, tail_row, carry)
            pltpu.sync_copy(obufs[0].at[pl.ds(0, sz)],
                            out_hbm.at[pl.ds(base, sz)])
            off += sz

    return pass2


def _finalize_tc(partials, gtarg, counts, weight, bias, mean_scale, b):
    wg, d2 = partials.shape[0] * partials.shape[1], partials.shape[2]
    d = d2 // 2

    def body(part_ref, gt_ref, cnt_ref, w_ref, b_ref, ms_ref, alpha_ref, beta_ref):
        gidx = lax.broadcasted_iota(jnp.int32, (wg, b), 1)
        onehot = (gt_ref[...] == gidx).astype(jnp.float32)
        part2d = part_ref[...].reshape(wg, d2)
        s = lax.dot_general(onehot, part2d, (((0,), (0,)), ((), ())),
                            preferred_element_type=jnp.float32)
        sx = s[:, :d]
        sx2 = s[:, d:]
        invn = 1.0 / cnt_ref[...]
        mean = sx * invn
        c = mean * ms_ref[...]
        var = sx2 * invn - 2.0 * c * mean + c * c
        alpha = w_ref[...] * lax.rsqrt(var + 1e-6)
        alpha_ref[...] = alpha
        beta_ref[...] = b_ref[...] - alpha * c

    return pl.pallas_call(
        body,
        out_shape=[
            jax.ShapeDtypeStruct((b, d), jnp.float32),
            jax.ShapeDtypeStruct((b, d), jnp.float32),
        ],
    )(partials, gtarg, counts, weight, bias, mean_scale)


def kernel(graph, tensor, batch_list, weight, bias, mean_scale):
    n, d = tensor.shape
    b = batch_list.shape[0]
    w = NC * NS
    rpw = n // w
    assert n == w * rpw and rpw % 8 == 0

    ends = jnp.cumsum(batch_list.astype(jnp.int32))
    ne_idx = jnp.where(batch_list > 0, jnp.arange(b, dtype=jnp.int32), b - 1)
    sufmin = lax.cummin(ne_idx[::-1])[::-1]
    nxt = jnp.concatenate([sufmin[1:], jnp.full((1,), b - 1, jnp.int32)])
    partials = _make_pass1(n, d, b, rpw, 224, 3)(tensor, ends, nxt)
    g0 = jnp.searchsorted(ends, jnp.arange(w, dtype=jnp.int32) * rpw,
                          side='right').astype(jnp.int32)
    g0 = jnp.minimum(g0 & ~7, b - GPAD)
    gtarg = (g0[:, None] + jnp.arange(GPAD, dtype=jnp.int32)[None, :]).reshape(-1, 1)
    counts = batch_list.astype(jnp.float32).reshape(b, 1)
    alpha, beta = _finalize_tc(
        partials, gtarg, counts,
        weight.reshape(1, d), bias.reshape(1, d), mean_scale.reshape(1, d), b)
    return _make_pass2(n, d, b, rpw, 176, 2)(tensor, ends, nxt, alpha, beta)


# speedup vs baseline: 1.5829x; 1.4286x over previous
import functools

import jax
import jax.numpy as jnp
from jax import lax
from jax.experimental import pallas as pl
from jax.experimental.pallas import tpu as pltpu
from jax.experimental.pallas import tpu_sc as plsc

NC = 2
NS = 16
L = 16
GPAD = 112
NJ = 8
BLK = 16


def _sload(ref, i):
    return ref[pl.ds(i, L)][0]


def _searchsorted_right(ends_v, row, b):
    def step(k, lo):
        sz = b >> (k + 1)
        probe = lo + sz - 1
        go = _sload(ends_v, probe) <= row
        return lo + jnp.where(go, sz, 0)

    return lax.fori_loop(0, b.bit_length() - 1, step, jnp.int32(0))


def _hop(g, e, row, ends_v, nxt_v):
    adv = row >= e
    gn = _sload(nxt_v, g)
    en = _sload(ends_v, gn)
    return jnp.where(adv, gn, g), jnp.where(adv, en, e)


def _make_pass1(n, d, b, rpw, chunk, ring):
    d2 = 2 * d
    nfull = (rpw // chunk) // ring * ring
    rem = rpw - nfull * chunk
    nblk = chunk // BLK
    assert chunk % BLK == 0 and rem % 8 == 0
    mesh = plsc.VectorSubcoreMesh(core_axis_name="c", subcore_axis_name="s")

    @functools.partial(
        pl.kernel,
        out_type=jax.ShapeDtypeStruct((NC * NS, GPAD, d2), jnp.float32),
        mesh=mesh,
        scratch_types=(
            [pltpu.VMEM((chunk, d), jnp.float32)] * ring
            + [
                pltpu.VMEM((GPAD, d2), jnp.float32),
                pltpu.VMEM((b + L,), jnp.int32),
                pltpu.VMEM((b + L,), jnp.int32),
                pltpu.VMEM((2 * L,), jnp.int32),
            ]
            + [pltpu.SemaphoreType.DMA] * ring
        ),
    )
    def pass1(tensor_hbm, ends_hbm, nxt_hbm, part_hbm, *scr):
        bufs = scr[:ring]
        acc, ends_v, nxt_v, ptr = scr[ring:ring + 4]
        sems = scr[ring + 4:]
        cid = lax.axis_index("c")
        sid = lax.axis_index("s")
        wid = cid * NS + sid
        zero16 = jnp.zeros((L,), jnp.float32)

        pltpu.sync_copy(ends_hbm, ends_v.at[pl.ds(0, b)])
        pltpu.sync_copy(nxt_hbm, nxt_v.at[pl.ds(0, b)])
        ptr[pl.ds(0, L)] = jnp.zeros((L,), jnp.int32)
        ptr[pl.ds(L, L)] = jnp.zeros((L,), jnp.int32)

        def arow(r, carry):
            for j in range(d2 // L):
                acc[r, pl.ds(L * j, L)] = zero16
            return carry

        lax.fori_loop(0, GPAD, arow, 0)

        row0 = wid * rpw
        gstart = _searchsorted_right(ends_v, row0, b)
        g0 = pl.multiple_of(jnp.minimum(gstart & ~7, b - GPAD), 8)

        def row_update(buf, rloc, row, gR, eR):
            gR, eR = _hop(gR, eR, row, ends_v, nxt_v)
            gl = jnp.minimum(gR - g0, GPAD - 1)
            for j in range(NJ):
                v = buf[rloc, pl.ds(L * j, L)]
                plsc.addupdate(acc.at[gl, pl.ds(L * j, L)], v)
                plsc.addupdate(acc.at[gl, pl.ds(d + L * j, L)], v * v)
            return gR, eR

        def blocks_body(buf, base, carry):
            def do_block(k, c2):
                gP, eP = c2
                rloc = k * BLK
                rb = base + rloc
                boundary = rb + BLK > eP
                glP = jnp.minimum(gP - g0, GPAD - 1)

                @pl.when(jnp.logical_not(boundary))
                def _fast():
                    for j in range(NJ):
                        bs = buf[rloc, pl.ds(L * j, L)]
                        bq = bs * bs
                        for r in range(1, BLK):
                            v = buf[rloc + r, pl.ds(L * j, L)]
                            bs = bs + v
                            bq = bq + v * v
                        plsc.addupdate(acc.at[glP, pl.ds(L * j, L)], bs)
                        plsc.addupdate(acc.at[glP, pl.ds(d + L * j, L)], bq)

                @pl.when(boundary)
                def _slow():
                    def srow(r, c3):
                        return row_update(buf, rloc + r, rb + r, c3[0], c3[1])

                    gR, eR = lax.fori_loop(0, BLK, srow, (gP, eP))
                    ptr[pl.ds(0, L)] = jnp.full((L,), gR, jnp.int32)
                    ptr[pl.ds(L, L)] = jnp.full((L,), eR, jnp.int32)

                gS = _sload(ptr, 0)
                eS = _sload(ptr, L)
                gP = jnp.where(boundary, gS, gP)
                eP = jnp.where(boundary, eS, eP)
                return _hop(gP, eP, rb + BLK, ends_v, nxt_v)

            return lax.fori_loop(0, nblk, do_block, carry)

        e0 = _sload(ends_v, gstart)
        carry = (gstart, e0)

        for i in range(ring):
            pltpu.async_copy(tensor_hbm.at[pl.ds(row0 + i * chunk, chunk)],
                             bufs[i], sems[i])

        def do_group(p, carry):
            base = row0 + ring * p * chunk
            for h in range(ring):
                pltpu.make_async_copy(tensor_hbm.at[pl.ds(0, chunk)],
                                      bufs[h], sems[h]).wait()

                @pl.when(ring * p + ring + h < nfull)
                def _prefetch(h=h, base=base):
                    pltpu.async_copy(
                        tensor_hbm.at[pl.ds(base + (ring + h) * chunk, chunk)],
                        bufs[h], sems[h])

                carry = blocks_body(bufs[h], base + h * chunk, carry)
            return carry

        carry = lax.fori_loop(0, nfull // ring, do_group, carry)
        off = 0
        while off < rem:
            sz = min(chunk, rem - off)
            base = row0 + nfull * chunk + off
            pltpu.sync_copy(tensor_hbm.at[pl.ds(base, sz)],
                            bufs[0].at[pl.ds(0, sz)])

            def tail_row(r, c2, base=base):
                return row_update(bufs[0], r, base + r, c2[0], c2[1])

            carry = lax.fori_loop(0, sz, tail_row, carry)
            off += sz

        pltpu.sync_copy(acc, part_hbm.at[wid])

    return pass1


def _make_pass2(n, d, b, rpw, chunk, ring):
    nfull = (rpw // chunk) // ring * ring
    rem = rpw - nfull * chunk
    nblk = chunk // BLK
    assert chunk % BLK == 0 and rem % 8 == 0
    mesh = plsc.VectorSubcoreMesh(core_axis_name="c", subcore_axis_name="s")

    @functools.partial(
        pl.kernel,
        out_type=jax.ShapeDtypeStruct((n, d), jnp.float32),
        mesh=mesh,
        scratch_types=(
            [pltpu.VMEM((chunk, d), jnp.float32)] * (2 * ring)
            + [
                pltpu.VMEM((GPAD, d), jnp.float32),
                pltpu.VMEM((GPAD, d), jnp.float32),
                pltpu.VMEM((b + L,), jnp.int32),
                pltpu.VMEM((b + L,), jnp.int32),
                pltpu.VMEM((2 * L,), jnp.int32),
            ]
            + [pltpu.SemaphoreType.DMA] * (2 * ring)
        ),
    )
    def pass2(tensor_hbm, ends_hbm, nxt_hbm, alpha_hbm, beta_hbm, out_hbm,
              *scr):
        bufs = scr[:ring]
        obufs = scr[ring:2 * ring]
        al, be, ends_v, nxt_v, ptr = scr[2 * ring:2 * ring + 5]
        sems = scr[2 * ring + 5:3 * ring + 5]
        osems = scr[3 * ring + 5:]
        cid = lax.axis_index("c")
        sid = lax.axis_index("s")
        wid = cid * NS + sid

        pltpu.sync_copy(ends_hbm, ends_v.at[pl.ds(0, b)])
        pltpu.sync_copy(nxt_hbm, nxt_v.at[pl.ds(0, b)])
        ptr[pl.ds(0, L)] = jnp.zeros((L,), jnp.int32)
        ptr[pl.ds(L, L)] = jnp.zeros((L,), jnp.int32)
        row0 = wid * rpw
        gstart = _searchsorted_right(ends_v, row0, b)
        g0c = pl.multiple_of(jnp.minimum(gstart & ~7, b - GPAD), 8)
        pltpu.sync_copy(alpha_hbm.at[pl.ds(g0c, GPAD)], al)
        pltpu.sync_copy(beta_hbm.at[pl.ds(g0c, GPAD)], be)

        def row_apply(buf, obuf, rloc, row, gR, eR):
            gR, eR = _hop(gR, eR, row, ends_v, nxt_v)
            gl = jnp.minimum(gR - g0c, GPAD - 1)
            for j in range(NJ):
                v = buf[rloc, pl.ds(L * j, L)]
                a = al[gl, pl.ds(L * j, L)]
                bb = be[gl, pl.ds(L * j, L)]
                obuf[rloc, pl.ds(L * j, L)] = v * a + bb
            return gR, eR

        def blocks_body(buf, obuf, base, carry):
            def do_block(k, c2):
                gP, eP = c2
                rloc = k * BLK
                rb = base + rloc
                boundary = rb + BLK > eP
                glP = jnp.minimum(gP - g0c, GPAD - 1)

                @pl.when(jnp.logical_not(boundary))
                def _fast():
                    a = [al[glP, pl.ds(L * j, L)] for j in range(NJ)]
                    bb = [be[glP, pl.ds(L * j, L)] for j in range(NJ)]
                    for r in range(BLK):
                        for j in range(NJ):
                            v = buf[rloc + r, pl.ds(L * j, L)]
                            obuf[rloc + r, pl.ds(L * j, L)] = v * a[j] + bb[j]

                @pl.when(boundary)
                def _slow():
                    def srow(r, c3):
                        return row_apply(buf, obuf, rloc + r, rb + r,
                                         c3[0], c3[1])

                    gR, eR = lax.fori_loop(0, BLK, srow, (gP, eP))
                    ptr[pl.ds(0, L)] = jnp.full((L,), gR, jnp.int32)
                    ptr[pl.ds(L, L)] = jnp.full((L,), eR, jnp.int32)

                gS = _sload(ptr, 0)
                eS = _sload(ptr, L)
                gP = jnp.where(boundary, gS, gP)
                eP = jnp.where(boundary, eS, eP)
                return _hop(gP, eP, rb + BLK, ends_v, nxt_v)

            return lax.fori_loop(0, nblk, do_block, carry)

        e0 = _sload(ends_v, gstart)
        carry = (gstart, e0)

        for i in range(ring):
            pltpu.async_copy(tensor_hbm.at[pl.ds(row0 + i * chunk, chunk)],
                             bufs[i], sems[i])

        def do_group(p, carry):
            base = row0 + ring * p * chunk
            for h in range(ring):
                cbase = base + h * chunk
                pltpu.make_async_copy(tensor_hbm.at[pl.ds(0, chunk)],
                                      bufs[h], sems[h]).wait()

                @pl.when(ring * p + ring + h < nfull)
                def _prefetch(h=h, base=base):
                    pltpu.async_copy(
                        tensor_hbm.at[pl.ds(base + (ring + h) * chunk, chunk)],
                        bufs[h], sems[h])

                @pl.when(p > 0)
                def _drain_out(h=h):
                    pltpu.make_async_copy(obufs[h],
                                          out_hbm.at[pl.ds(0, chunk)],
                                          osems[h]).wait()

                carry = blocks_body(bufs[h], obufs[h], cbase, carry)
                pltpu.async_copy(obufs[h], out_hbm.at[pl.ds(cbase, chunk)],
                                 osems[h])
            return carry

        carry = lax.fori_loop(0, nfull // ring, do_group, carry)
        for h in range(ring):
            pltpu.make_async_copy(obufs[h], out_hbm.at[pl.ds(0, chunk)],
                                  osems[h]).wait()
        off = 0
        while off < rem:
            sz = min(chunk, rem - off)
            base = row0 + nfull * chunk + off
            pltpu.sync_copy(tensor_hbm.at[pl.ds(base, sz)],
                            bufs[0].at[pl.ds(0, sz)])

            def tail_row(r, c2, base=base):
                return row_apply(bufs[0], obufs[0], r, base + r, c2[0], c2[1])

            carry = lax.fori_loop(0, sz, tail_row, carry)
            pltpu.sync_copy(obufs[0].at[pl.ds(0, sz)],
                            out_hbm.at[pl.ds(base, sz)])
            off += sz

    return pass2


def _finalize_tc(partials, gtarg, counts, weight, bias, mean_scale, b):
    wg, d2 = partials.shape[0] * partials.shape[1], partials.shape[2]
    d = d2 // 2

    def body(part_ref, gt_ref, cnt_ref, w_ref, b_ref, ms_ref, alpha_ref, beta_ref):
        gidx = lax.broadcasted_iota(jnp.int32, (wg, b), 1)
        onehot = (gt_ref[...] == gidx).astype(jnp.float32)
        part2d = part_ref[...].reshape(wg, d2)
        s = lax.dot_general(onehot, part2d, (((0,), (0,)), ((), ())),
                            preferred_element_type=jnp.float32)
        sx = s[:, :d]
        sx2 = s[:, d:]
        invn = 1.0 / cnt_ref[...]
        mean = sx * invn
        c = mean * ms_ref[...]
        var = sx2 * invn - 2.0 * c * mean + c * c
        alpha = w_ref[...] * lax.rsqrt(var + 1e-6)
        alpha_ref[...] = alpha
        beta_ref[...] = b_ref[...] - alpha * c

    return pl.pallas_call(
        body,
        out_shape=[
            jax.ShapeDtypeStruct((b, d), jnp.float32),
            jax.ShapeDtypeStruct((b, d), jnp.float32),
        ],
    )(partials, gtarg, counts, weight, bias, mean_scale)


def kernel(graph, tensor, batch_list, weight, bias, mean_scale):
    n, d = tensor.shape
    b = batch_list.shape[0]
    w = NC * NS
    rpw = n // w
    assert n == w * rpw and rpw % 8 == 0

    ends = jnp.cumsum(batch_list.astype(jnp.int32))
    ne_idx = jnp.where(batch_list > 0, jnp.arange(b, dtype=jnp.int32), b - 1)
    sufmin = lax.cummin(ne_idx[::-1])[::-1]
    nxt = jnp.concatenate([sufmin[1:], jnp.full((1,), b - 1, jnp.int32)])
    partials = _make_pass1(n, d, b, rpw, 144, 2)(tensor, ends, nxt)
    g0 = jnp.searchsorted(ends, jnp.arange(w, dtype=jnp.int32) * rpw,
                          side='right').astype(jnp.int32)
    g0 = jnp.minimum(g0 & ~7, b - GPAD)
    gtarg = (g0[:, None] + jnp.arange(GPAD, dtype=jnp.int32)[None, :]).reshape(-1, 1)
    counts = batch_list.astype(jnp.float32).reshape(b, 1)
    alpha, beta = _finalize_tc(
        partials, gtarg, counts,
        weight.reshape(1, d), bias.reshape(1, d), mean_scale.reshape(1, d), b)
    return _make_pass2(n, d, b, rpw, 144, 2)(tensor, ends, nxt, alpha, beta)
